# Initial kernel scaffold; baseline (speedup 1.0000x reference)
import jax
import jax.numpy as jnp
from jax.experimental import pallas as pl


def kernel(sub, rel, obj, ent_emb, rel_emb):
    raise NotImplementedError("write your pallas kernel here")


# speedup vs baseline: 1.5401x; 1.5401x over previous
import functools

import jax
import jax.numpy as jnp
from jax import lax
from jax.experimental import pallas as pl
from jax.experimental.pallas import tpu as pltpu
from jax.experimental.pallas import tpu_sc as plsc

NUM_ENT = 1000000
NUM_REL = 1000
D = 64
MARGIN = 12.0
BATCH = 4096
ERANGE = (MARGIN + 2.0) / D
PI = 3.141592653589793

NC, NS, L = 2, 16, 16
NW = NC * NS
BPW = BATCH // NW


def _trig_body(rel_ref, out_ref):
    ph = rel_ref[...] * (PI / ERANGE)
    out_ref[...] = jnp.concatenate([jnp.cos(ph), jnp.sin(ph)], axis=1)


_trig_call = pl.pallas_call(
    _trig_body,
    out_shape=jax.ShapeDtypeStruct((NUM_REL, 2 * D), jnp.float32),
)

_sc_mesh = plsc.VectorSubcoreMesh(core_axis_name="c", subcore_axis_name="s")


@functools.partial(
    pl.kernel,
    out_type=jax.ShapeDtypeStruct((BATCH,), jnp.float32),
    mesh=_sc_mesh,
    compiler_params=pltpu.CompilerParams(needs_layout_passes=False),
    scratch_types=[
        pltpu.VMEM((BPW,), jnp.int32),
        pltpu.VMEM((BPW,), jnp.int32),
        pltpu.VMEM((BPW,), jnp.int32),
        pltpu.VMEM((BPW, 2 * D), jnp.float32),
        pltpu.VMEM((BPW, 2 * D), jnp.float32),
        pltpu.VMEM((BPW, 2 * D), jnp.float32),
        pltpu.VMEM((BPW,), jnp.float32),
        pltpu.SemaphoreType.DMA,
        pltpu.SemaphoreType.DMA,
        pltpu.SemaphoreType.DMA,
    ],
)
def _sc_score(sub_hbm, rel_hbm, obj_hbm, ent_hbm, trig_hbm, out_hbm,
              sub_v, obj_v, rel_v, h_v, t_v, r_v, o_v, sem_h, sem_t, sem_r):
    wid = lax.axis_index("s") * NC + lax.axis_index("c")
    base = wid * BPW
    pltpu.sync_copy(sub_hbm.at[pl.ds(base, BPW)], sub_v)
    pltpu.sync_copy(obj_hbm.at[pl.ds(base, BPW)], obj_v)
    pltpu.sync_copy(rel_hbm.at[pl.ds(base, BPW)], rel_v)
    ch = pltpu.async_copy(ent_hbm.at[sub_v], h_v, sem_h)
    ct = pltpu.async_copy(ent_hbm.at[obj_v], t_v, sem_t)
    cr = pltpu.async_copy(trig_hbm.at[rel_v], r_v, sem_r)
    ch.wait()
    ct.wait()
    cr.wait()

    lane = lax.iota(jnp.int32, L)

    def gbody(g, carry):
        vec = jnp.zeros((L,), jnp.float32)
        for e in range(L):
            b = g * L + e
            acc = jnp.zeros((L,), jnp.float32)
            for k in range(D // L):
                sl = pl.ds(k * L, L)
                sl2 = pl.ds(D + k * L, L)
                re_h = h_v[b, sl]
                im_h = h_v[b, sl2]
                cs = r_v[b, sl]
                sn = r_v[b, sl2]
                re_t = t_v[b, sl]
                im_t = t_v[b, sl2]
                re_s = re_h * cs - im_h * sn
                im_s = re_h * sn + im_h * cs
                acc = acc + jnp.abs(re_s - re_t) + jnp.abs(im_s - im_t)
            vec = jnp.where(lane == e, jnp.sum(acc), vec)
        o_v[pl.ds(g * L, L)] = MARGIN - vec
        return carry

    lax.fori_loop(0, BPW // L, gbody, 0)
    pltpu.sync_copy(o_v, out_hbm.at[pl.ds(base, BPW)])


def kernel(sub, rel, obj, ent_emb, rel_emb):
    trig = _trig_call(rel_emb)
    return _sc_score(sub.astype(jnp.int32), rel.astype(jnp.int32),
                     obj.astype(jnp.int32), ent_emb, trig)
